# baseline (speedup 1.0000x reference)
import functools

import jax
import jax.numpy as jnp
from jax import lax
from jax.experimental import pallas as pl
from jax.experimental.pallas import tpu as pltpu
from jax.experimental.pallas import tpu_sc as plsc

VOCAB = 1000000
EMBED_DIM = 32
BATCH = 4096
HIST_LEN = 50
NC = 2
NS = 16
NW = NC * NS
BPW = BATCH // NW
L = 16


@functools.cache
def _make_kernel():
    mesh = plsc.VectorSubcoreMesh(core_axis_name="c", subcore_axis_name="s")

    @functools.partial(
        pl.kernel,
        mesh=mesh,
        out_type=jax.ShapeDtypeStruct((HIST_LEN, EMBED_DIM, BATCH), jnp.float32),
        scratch_types=[
            pltpu.VMEM((HIST_LEN, BPW), jnp.int32),
            pltpu.VMEM((EMBED_DIM, BPW), jnp.float32),
            pltpu.VMEM((EMBED_DIM, BPW), jnp.float32),
            pltpu.SemaphoreType.DMA,
            pltpu.SemaphoreType.DMA,
            pltpu.SemaphoreType.DMA,
            pltpu.SemaphoreType.DMA,
        ],
        compiler_params=pltpu.CompilerParams(
            use_tc_tiling_on_sc=False, needs_layout_passes=False),
    )
    def emb_kernel(tbl_hbm, idx_hbm, out_hbm, idx_v,
                   gbuf0, gbuf1, gsem0, gsem1, ssem0, ssem1):
        gbufs = (gbuf0, gbuf1)
        gsems = (gsem0, gsem1)
        ssems = (ssem0, ssem1)
        wid = lax.axis_index("s") * NC + lax.axis_index("c")
        pltpu.sync_copy(idx_hbm.at[wid], idx_v)
        b0 = wid * BPW

        def fire_gathers(j, b):
            for f in range(EMBED_DIM):
                pltpu.async_copy(
                    tbl_hbm.at[f].at[idx_v.at[j]], gbufs[b].at[f], gsems[b])

        def drain_gathers(b):
            pltpu.make_async_copy(
                out_hbm.at[0, :, pl.ds(0, BPW)], gbufs[b], gsems[b]).wait()

        def fire_store(j, b):
            pltpu.async_copy(
                gbufs[b], out_hbm.at[j, :, pl.ds(b0, BPW)], ssems[b])

        def wait_store(b):
            pltpu.make_async_copy(
                gbufs[b], out_hbm.at[0, :, pl.ds(0, BPW)], ssems[b]).wait()

        fire_gathers(0, 0)
        fire_gathers(1, 1)
        drain_gathers(0)
        fire_store(0, 0)

        @pl.loop(2, HIST_LEN, step=2)
        def _steady(o):
            for i in range(2):
                j = o + i
                b = i
                wait_store(b)
                fire_gathers(j, b)
                drain_gathers(1 - b)
                fire_store(j - 1, 1 - b)

        drain_gathers(1)
        fire_store(HIST_LEN - 1, 1)
        wait_store(0)
        wait_store(1)

    return emb_kernel


def kernel(input, emb_table):
    idx = (input.astype(jnp.int32)
           .reshape(NW, BPW, HIST_LEN)
           .transpose(0, 2, 1))
    out = _make_kernel()(emb_table.T, idx)
    return out.transpose(2, 0, 1)


# speedup vs baseline: 7.6904x; 7.6904x over previous
import functools

import jax
import jax.numpy as jnp
from jax import lax
from jax.experimental import pallas as pl
from jax.experimental.pallas import tpu as pltpu
from jax.experimental.pallas import tpu_sc as plsc

VOCAB = 1000000
EMBED_DIM = 32
BATCH = 4096
HIST_LEN = 50
NC = 2
NS = 16
NW = NC * NS
BPW = BATCH // NW
L = 16
VPAD = 1000064
NTILES = VPAD // 128
TOT_WORDS = EMBED_DIM * VPAD


@functools.cache
def _make_kernel():
    mesh = plsc.VectorSubcoreMesh(core_axis_name="c", subcore_axis_name="s")

    @functools.partial(
        pl.kernel,
        mesh=mesh,
        out_type=jax.ShapeDtypeStruct((HIST_LEN, EMBED_DIM, BATCH), jnp.float32),
        scratch_types=[
            pltpu.VMEM((HIST_LEN, BPW), jnp.int32),
            pltpu.VMEM((EMBED_DIM, BPW), jnp.int32),
            pltpu.VMEM((EMBED_DIM, BPW), jnp.int32),
            pltpu.VMEM((EMBED_DIM, BPW), jnp.float32),
            pltpu.VMEM((EMBED_DIM, BPW), jnp.float32),
            pltpu.SemaphoreType.DMA,
            pltpu.SemaphoreType.DMA,
            pltpu.SemaphoreType.DMA,
            pltpu.SemaphoreType.DMA,
        ],
        compiler_params=pltpu.CompilerParams(
            use_tc_tiling_on_sc=False, needs_layout_passes=False),
    )
    def emb_kernel(tbl_hbm, idx_hbm, out_hbm, idx_v, ibuf0, ibuf1,
                   gbuf0, gbuf1, gsem0, gsem1, ssem0, ssem1):
        ibufs = (ibuf0, ibuf1)
        gbufs = (gbuf0, gbuf1)
        gsems = (gsem0, gsem1)
        ssems = (ssem0, ssem1)
        wid = lax.axis_index("s") * NC + lax.axis_index("c")
        pltpu.sync_copy(idx_hbm.at[wid], idx_v)
        b0 = wid * BPW

        def build_indices(j, b):
            for k in range(BPW // L):
                v = idx_v[j, pl.ds(k * L, L)]
                tv = ((v >> 7) << 10) + (v & 127)
                for f in range(EMBED_DIM):
                    off = (f // 8) * (NTILES * 1024) + (f % 8) * 128
                    ibufs[b][f, pl.ds(k * L, L)] = tv + off

        def fire_gathers(j, b):
            del j
            for f in range(EMBED_DIM):
                pltpu.async_copy(
                    tbl_hbm.at[ibufs[b].at[f]], gbufs[b].at[f], gsems[b])

        def drain_gathers(b):
            pltpu.make_async_copy(
                out_hbm.at[0, :, pl.ds(0, BPW)], gbufs[b], gsems[b]).wait()

        def fire_store(j, b):
            pltpu.async_copy(
                gbufs[b], out_hbm.at[j, :, pl.ds(b0, BPW)], ssems[b])

        def wait_store(b):
            pltpu.make_async_copy(
                gbufs[b], out_hbm.at[0, :, pl.ds(0, BPW)], ssems[b]).wait()

        build_indices(0, 0)
        fire_gathers(0, 0)
        build_indices(1, 1)
        fire_gathers(1, 1)
        drain_gathers(0)
        fire_store(0, 0)

        @pl.loop(2, HIST_LEN, step=2)
        def _steady(o):
            for i in range(2):
                j = o + i
                b = i
                build_indices(j, b)
                wait_store(b)
                fire_gathers(j, b)
                drain_gathers(1 - b)
                fire_store(j - 1, 1 - b)

        drain_gathers(1)
        fire_store(HIST_LEN - 1, 1)
        wait_store(0)
        wait_store(1)

    return emb_kernel


def kernel(input, emb_table):
    tp = jnp.pad(emb_table.T, ((0, 0), (0, VPAD - VOCAB)))
    tflat = (tp.reshape(EMBED_DIM // 8, 8, NTILES, 128)
             .transpose(0, 2, 1, 3)
             .reshape(-1))
    idx = (input.astype(jnp.int32)
           .reshape(NW, BPW, HIST_LEN)
           .transpose(0, 2, 1))
    out = _make_kernel()(tflat, idx)
    return out.transpose(2, 0, 1)
